# baseline (speedup 1.0000x reference)
import functools

import jax
import jax.numpy as jnp
from jax import lax
from jax.experimental import pallas as pl
from jax.experimental.pallas import tpu as pltpu
from jax.experimental.pallas import tpu_sc as plsc

_NW = 32
_RPC = 16


def _sc_lookup(indices, flat_tab, F, V, E, B):
    n_chunks = B // _RPC
    per_w = n_chunks // _NW
    row_w = F * E
    mesh = plsc.VectorSubcoreMesh(core_axis_name="c", subcore_axis_name="s")

    @functools.partial(
        pl.kernel,
        mesh=mesh,
        compiler_params=pltpu.CompilerParams(needs_layout_passes=False),
        out_type=jax.ShapeDtypeStruct((B, row_w), jnp.float32),
        scratch_types=[
            pltpu.VMEM((F * V * E,), jnp.float32),
            pltpu.VMEM((_RPC, F), jnp.int32),
            pltpu.VMEM((_RPC, F), jnp.int32),
            pltpu.VMEM((_RPC, row_w), jnp.float32),
            pltpu.VMEM((_RPC, row_w), jnp.float32),
            pltpu.SemaphoreType.DMA,
            pltpu.SemaphoreType.DMA,
        ],
    )
    def k(idx_hbm, tab_hbm, out_hbm, tab_v, idx_v0, idx_v1, out_v0, out_v1,
          sem0, sem1):
        wid = lax.axis_index("s") * 2 + lax.axis_index("c")
        pltpu.sync_copy(tab_hbm, tab_v)
        lanes = lax.iota(jnp.int32, 16)
        idx_bufs = (idx_v0, idx_v1)
        out_bufs = (out_v0, out_v1)
        sems = (sem0, sem1)

        def chunk_body(g, carry):
            for u in range(2):
                chunk = wid * per_w + 2 * g + u
                r0 = chunk * _RPC
                pltpu.sync_copy(idx_hbm.at[pl.ds(r0, _RPC), :], idx_bufs[u])

                @pl.when(g > 0)
                def _wait_prev():
                    pltpu.make_async_copy(
                        out_bufs[u], out_hbm.at[pl.ds(r0, _RPC), :], sems[u]
                    ).wait()

                def b_body(b, carry2, u=u):
                    bvec = jnp.full((16,), b, jnp.int32)
                    for f in range(F):
                        idv = plsc.load_gather(
                            idx_bufs[u],
                            [bvec, jnp.full((16,), f, jnp.int32)],
                        )
                        src0 = (idv + f * V) * E + lanes
                        for e0 in (0, 16, 32, 34):
                            w = plsc.load_gather(tab_v, [src0 + e0])
                            out_bufs[u][b, pl.ds(f * E + e0, 16)] = w
                    return carry2

                lax.fori_loop(0, _RPC, b_body, 0)
                pltpu.async_copy(
                    out_bufs[u], out_hbm.at[pl.ds(r0, _RPC), :], sems[u]
                )
            return carry

        lax.fori_loop(0, per_w // 2, chunk_body, 0)
        for u in range(2):
            last = wid * per_w + per_w - 2 + u
            pltpu.make_async_copy(
                out_bufs[u],
                out_hbm.at[pl.ds(last * _RPC, _RPC), :],
                sems[u],
            ).wait()

    return k(indices, flat_tab)


def kernel(indices, tables):
    F, V, E = tables.shape
    B = indices.shape[0]
    flat_tab = tables.reshape(F * V * E)
    return _sc_lookup(indices, flat_tab, F, V, E, B)
